# baseline (speedup 1.0000x reference)
import functools
import math

import jax
import jax.numpy as jnp
import numpy as np
from jax import lax
from jax.experimental import pallas as pl
from jax.experimental.pallas import tpu as pltpu
from jax.experimental.pallas import tpu_sc as plsc

_SQRT3 = math.sqrt(3.0)
_SH_NORM = math.sqrt(5.0 / (4.0 * math.pi))

_NC = 2
_NS = 16
_LANE = 128
_CHUNK_ROWS = 8


def _change_mat_np():
    s2 = 2 ** (-0.5)
    s3 = 3 ** (-0.5)
    s6 = 6 ** (-0.5)
    return np.array([
        [s3, 0, 0, 0, s3, 0, 0, 0, s3],
        [0, 0, 0, 0, 0, s2, 0, -s2, 0],
        [0, 0, -s2, 0, 0, 0, s2, 0, 0],
        [0, s2, 0, -s2, 0, 0, 0, 0, 0],
        [0, 0, 0.5 ** 0.5, 0, 0, 0, 0.5 ** 0.5, 0, 0],
        [0, s2, 0, s2, 0, 0, 0, 0, 0],
        [-s6, 0, 0, 0, 2 * s6, 0, 0, 0, -s6],
        [0, 0, 0, 0, 0, s2, 0, s2, 0],
        [-s2, 0, 0, 0, 0, 0, 0, 0, s2],
    ], dtype=np.float32)


def _edge_body(x_ref, vt_ref, ws1_ref, bs1_ref, w2_ref, wi1_ref, bi1_ref,
               b2_ref, out_ref, *, nreal, off):
    x = x_ref[...].astype(jnp.bfloat16)
    h1 = jnp.dot(x, ws1_ref[...].astype(jnp.bfloat16),
                 preferred_element_type=jnp.float32) + bs1_ref[...]
    h1 = h1 * (0.5 * jnp.tanh(0.5 * h1) + 0.5)
    es = jnp.sum(h1 * w2_ref[0:1, :], axis=1, keepdims=True) + b2_ref[0:1, 0:1]
    h2 = jnp.dot(x, wi1_ref[...].astype(jnp.bfloat16),
                 preferred_element_type=jnp.float32) + bi1_ref[...]
    h2 = h2 * (0.5 * jnp.tanh(0.5 * h2) + 0.5)
    ei = jnp.sum(h2 * w2_ref[1:2, :], axis=1, keepdims=True) + b2_ref[0:1, 1:2]
    esei_t = jnp.concatenate([es, ei], axis=1).T

    vt = vt_ref[...]
    vx, vy, vz = vt[0:1, :], vt[1:2, :], vt[2:3, :]
    r = jnp.sqrt(vx * vx + vy * vy + vz * vz)
    rinv = 1.0 / jnp.maximum(r, 1e-12)
    ux, uy, uz = vx * rinv, vy * rinv, vz * rinv
    eis = esei_t[1:2, :] * _SH_NORM
    sh0 = (_SQRT3 * ux * uz) * eis
    sh1 = (_SQRT3 * ux * uy) * eis
    sh2 = (uy * uy - 0.5 * (ux * ux + uz * uz)) * eis
    sh3 = (_SQRT3 * uy * uz) * eis
    sh4 = ((_SQRT3 / 2.0) * (uz * uz - ux * ux)) * eis

    one = jnp.ones_like(eis)
    zero = jnp.zeros_like(eis)
    out_t = jnp.concatenate(
        [esei_t[0:1, :], sh0, sh1, sh2, sh3, sh4, one, zero], axis=0)
    valid = (pl.program_id(0) + off < nreal).astype(jnp.float32)
    out_ref[...] = out_t.T * valid


def _scatter_body(vals_hbm, idx_hbm, zeros_hbm, out_hbm, idx_v, vals_v, acc,
                  sem, *, n_pad, rows_per_worker):
    c = lax.axis_index("c")
    s = lax.axis_index("s")
    stripe = n_pad // _NS
    pltpu.sync_copy(zeros_hbm.at[pl.ds(s * stripe, stripe)],
                    acc.at[pl.ds(s * stripe, stripe)])
    plsc.subcore_barrier()
    wid = c * _NS + s
    base = wid * rows_per_worker
    nchunks = rows_per_worker // _CHUNK_ROWS

    def chunk(i, carry):
        row = base + i * _CHUNK_ROWS
        pltpu.sync_copy(idx_hbm.at[pl.ds(row, _CHUNK_ROWS)], idx_v)
        pltpu.sync_copy(vals_hbm.at[pl.ds(row, _CHUNK_ROWS)], vals_v)
        cps = [pltpu.async_copy(vals_v.at[j], acc.at[idx_v.at[j]], sem, add=True)
               for j in range(_CHUNK_ROWS)]
        for cp in cps:
            cp.wait()
        return carry

    lax.fori_loop(0, nchunks, chunk, 0)
    plsc.subcore_barrier()
    pltpu.sync_copy(acc.at[pl.ds(s * stripe, stripe)],
                    out_hbm.at[c, pl.ds(s * stripe, stripe)])


def _finish_body(p0_ref, p1_ref, bi_ref, cm_ref, out_ref, *, n_pad, b):
    accm = (p0_ref[0] + p0_ref[1]) + (p1_ref[0] + p1_ref[1])
    cnt = accm[:, 6:7]
    nv = accm[:, 0:6] / jnp.maximum(cnt, 1.0)
    ones = jnp.ones((n_pad, 1), jnp.float32)
    zeros = jnp.zeros((n_pad, 1), jnp.float32)
    nv8 = jnp.concatenate([nv, ones, zeros], axis=1)
    bi = bi_ref[...]
    rows = lax.broadcasted_iota(jnp.int32, (b, n_pad), 0)
    oh = (rows == bi).astype(jnp.float32)
    seg = jnp.dot(oh, nv8, preferred_element_type=jnp.float32)
    nb = jnp.maximum(seg[:, 6:7], 1.0)
    g = seg[:, 0:6] / nb
    flat = jnp.concatenate(
        [g[:, 0:1], jnp.zeros((b, 3), jnp.float32), g[:, 1:6]], axis=1)
    out_ref[...] = jnp.dot(flat, cm_ref[...], preferred_element_type=jnp.float32)


def kernel(x_edge, edge_vec, idx_t, batch_idx, batch_size,
           Ws1, bs1, Ws2, bs2, Wi1, bi1, Wi2, bi2):
    E, D = x_edge.shape
    N = batch_idx.shape[0]
    B = 16

    blk = 2560
    nreal = E // blk
    chunk_edges = _LANE * _CHUNK_ROWS
    e_pad = ((E + _NC * _NS * chunk_edges - 1)
             // (_NC * _NS * chunk_edges)) * (_NC * _NS * chunk_edges)
    nblk = e_pad // blk
    super_rows = e_pad // _LANE
    rows_per_worker = super_rows // (_NC * _NS)
    n_pad = ((N + _NS * 16 - 1) // (_NS * 16)) * (_NS * 16)

    w2 = jnp.concatenate([Ws2.reshape(1, D), Wi2.reshape(1, D)], axis=0)
    b2 = jnp.concatenate([bs2.reshape(1, 1), bi2.reshape(1, 1)], axis=1)

    nhalf = nblk // 2
    rows_half = super_rows // 2
    rows_per_worker = rows_half // (_NC * _NS)

    vt = edge_vec.T
    bs1r, bi1r = bs1.reshape(1, D), bi1.reshape(1, D)

    def stage1(off):
        return pl.pallas_call(
            functools.partial(_edge_body, nreal=nreal, off=off),
            grid=(nhalf,),
            in_specs=[
                pl.BlockSpec((blk, D), lambda i: (jnp.minimum(i + off, nreal - 1), 0)),
                pl.BlockSpec((3, blk), lambda i: (0, jnp.minimum(i + off, nreal - 1))),
                pl.BlockSpec((D, D), lambda i: (0, 0)),
                pl.BlockSpec((1, D), lambda i: (0, 0)),
                pl.BlockSpec((2, D), lambda i: (0, 0)),
                pl.BlockSpec((D, D), lambda i: (0, 0)),
                pl.BlockSpec((1, D), lambda i: (0, 0)),
                pl.BlockSpec((1, 2), lambda i: (0, 0)),
            ],
            out_specs=pl.BlockSpec((blk, 8), lambda i: (i, 0)),
            out_shape=jax.ShapeDtypeStruct((nhalf * blk, 8), jnp.float32),
        )(x_edge, vt, Ws1, bs1r, w2, Wi1, bi1r, b2)

    pad_n = e_pad - E
    idx_pad = jnp.concatenate(
        [idx_t, (jnp.arange(pad_n, dtype=jnp.int32) % N)])
    idx2 = idx_pad.reshape(super_rows, _LANE)
    zeros_acc = jnp.zeros((n_pad, 8), jnp.float32)

    mesh = plsc.VectorSubcoreMesh(core_axis_name="c", subcore_axis_name="s")
    scatter = pl.kernel(
        functools.partial(_scatter_body, n_pad=n_pad,
                          rows_per_worker=rows_per_worker),
        out_type=jax.ShapeDtypeStruct((_NC, n_pad, 8), jnp.float32),
        mesh=mesh,
        compiler_params=pltpu.CompilerParams(use_tc_tiling_on_sc=False),
        scratch_types=[
            pltpu.VMEM((_CHUNK_ROWS, _LANE), jnp.int32),
            pltpu.VMEM((_CHUNK_ROWS, _LANE, 8), jnp.float32),
            pltpu.VMEM_SHARED((n_pad, 8), jnp.float32),
            pltpu.SemaphoreType.DMA,
        ],
    )

    vals0 = stage1(0)
    p0 = scatter(vals0.reshape(rows_half, _LANE, 8), idx2[:rows_half], zeros_acc)
    vals1 = stage1(nhalf)
    p1 = scatter(vals1.reshape(rows_half, _LANE, 8), idx2[rows_half:], zeros_acc)

    bi_pad = jnp.concatenate(
        [batch_idx, jnp.full((n_pad - N,), B, jnp.int32)]).reshape(1, n_pad)
    cm = jnp.asarray(_change_mat_np())
    stress = pl.pallas_call(
        functools.partial(_finish_body, n_pad=n_pad, b=B),
        out_shape=jax.ShapeDtypeStruct((B, 9), jnp.float32),
    )(p0, p1, bi_pad, cm)
    return stress.reshape(B, 3, 3)


# speedup vs baseline: 1.0047x; 1.0047x over previous
import functools
import math

import jax
import jax.numpy as jnp
import numpy as np
from jax import lax
from jax.experimental import pallas as pl
from jax.experimental.pallas import tpu as pltpu
from jax.experimental.pallas import tpu_sc as plsc

_SQRT3 = math.sqrt(3.0)
_SH_NORM = math.sqrt(5.0 / (4.0 * math.pi))

_NC = 2
_NS = 16
_LANE = 128
_CHUNK_ROWS = 8


def _change_mat_np():
    s2 = 2 ** (-0.5)
    s3 = 3 ** (-0.5)
    s6 = 6 ** (-0.5)
    return np.array([
        [s3, 0, 0, 0, s3, 0, 0, 0, s3],
        [0, 0, 0, 0, 0, s2, 0, -s2, 0],
        [0, 0, -s2, 0, 0, 0, s2, 0, 0],
        [0, s2, 0, -s2, 0, 0, 0, 0, 0],
        [0, 0, 0.5 ** 0.5, 0, 0, 0, 0.5 ** 0.5, 0, 0],
        [0, s2, 0, s2, 0, 0, 0, 0, 0],
        [-s6, 0, 0, 0, 2 * s6, 0, 0, 0, -s6],
        [0, 0, 0, 0, 0, s2, 0, s2, 0],
        [-s2, 0, 0, 0, 0, 0, 0, 0, s2],
    ], dtype=np.float32)


def _edge_body(x_ref, vt_ref, ws1_ref, bs1_ref, w2_ref, wi1_ref, bi1_ref,
               b2_ref, out_ref, *, nreal, off):
    x = x_ref[...]
    h1 = jnp.dot(x, ws1_ref[...], preferred_element_type=jnp.float32) + bs1_ref[...]
    h1 = h1 * (0.5 * jnp.tanh(0.5 * h1) + 0.5)
    es = jnp.sum(h1 * w2_ref[0:1, :], axis=1, keepdims=True) + b2_ref[0:1, 0:1]
    h2 = jnp.dot(x, wi1_ref[...], preferred_element_type=jnp.float32) + bi1_ref[...]
    h2 = h2 * (0.5 * jnp.tanh(0.5 * h2) + 0.5)
    ei = jnp.sum(h2 * w2_ref[1:2, :], axis=1, keepdims=True) + b2_ref[0:1, 1:2]
    esei_t = jnp.concatenate([es, ei], axis=1).T

    vt = vt_ref[...]
    vx, vy, vz = vt[0:1, :], vt[1:2, :], vt[2:3, :]
    r = jnp.sqrt(vx * vx + vy * vy + vz * vz)
    rinv = 1.0 / jnp.maximum(r, 1e-12)
    ux, uy, uz = vx * rinv, vy * rinv, vz * rinv
    eis = esei_t[1:2, :] * _SH_NORM
    sh0 = (_SQRT3 * ux * uz) * eis
    sh1 = (_SQRT3 * ux * uy) * eis
    sh2 = (uy * uy - 0.5 * (ux * ux + uz * uz)) * eis
    sh3 = (_SQRT3 * uy * uz) * eis
    sh4 = ((_SQRT3 / 2.0) * (uz * uz - ux * ux)) * eis

    one = jnp.ones_like(eis)
    zero = jnp.zeros_like(eis)
    out_t = jnp.concatenate(
        [esei_t[0:1, :], sh0, sh1, sh2, sh3, sh4, one, zero], axis=0)
    valid = (pl.program_id(0) + off < nreal).astype(jnp.float32)
    out_ref[...] = out_t.T * valid


def _scatter_body(vals_hbm, idx_hbm, zeros_hbm, out_hbm, idx_v, vals_v, acc,
                  sem, *, n_pad, rows_per_worker):
    c = lax.axis_index("c")
    s = lax.axis_index("s")
    stripe = n_pad // _NS
    pltpu.sync_copy(zeros_hbm.at[pl.ds(s * stripe, stripe)],
                    acc.at[pl.ds(s * stripe, stripe)])
    plsc.subcore_barrier()
    wid = c * _NS + s
    base = wid * rows_per_worker
    nchunks = rows_per_worker // _CHUNK_ROWS

    def chunk(i, carry):
        row = base + i * _CHUNK_ROWS
        pltpu.sync_copy(idx_hbm.at[pl.ds(row, _CHUNK_ROWS)], idx_v)
        pltpu.sync_copy(vals_hbm.at[pl.ds(row * _LANE, _CHUNK_ROWS * _LANE)],
                        vals_v)
        cps = [pltpu.async_copy(vals_v.at[pl.ds(j * _LANE, _LANE)],
                                acc.at[idx_v.at[j]], sem, add=True)
               for j in range(_CHUNK_ROWS)]
        for cp in cps:
            cp.wait()
        return carry

    lax.fori_loop(0, nchunks, chunk, 0)
    plsc.subcore_barrier()
    pltpu.sync_copy(acc.at[pl.ds(s * stripe, stripe)],
                    out_hbm.at[c, pl.ds(s * stripe, stripe)])


def _finish_body(p0_ref, p1_ref, bi_ref, cm_ref, out_ref, *, n_pad, b):
    accm = (p0_ref[0] + p0_ref[1]) + (p1_ref[0] + p1_ref[1])
    cnt = accm[:, 6:7]
    nv = accm[:, 0:6] / jnp.maximum(cnt, 1.0)
    ones = jnp.ones((n_pad, 1), jnp.float32)
    zeros = jnp.zeros((n_pad, 1), jnp.float32)
    nv8 = jnp.concatenate([nv, ones, zeros], axis=1)
    bi = bi_ref[...]
    rows = lax.broadcasted_iota(jnp.int32, (b, n_pad), 0)
    oh = (rows == bi).astype(jnp.float32)
    seg = jnp.dot(oh, nv8, preferred_element_type=jnp.float32)
    nb = jnp.maximum(seg[:, 6:7], 1.0)
    g = seg[:, 0:6] / nb
    flat = jnp.concatenate(
        [g[:, 0:1], jnp.zeros((b, 3), jnp.float32), g[:, 1:6]], axis=1)
    out_ref[...] = jnp.dot(flat, cm_ref[...], preferred_element_type=jnp.float32)


def kernel(x_edge, edge_vec, idx_t, batch_idx, batch_size,
           Ws1, bs1, Ws2, bs2, Wi1, bi1, Wi2, bi2):
    E, D = x_edge.shape
    N = batch_idx.shape[0]
    B = 16

    blk = 2560
    nreal = E // blk
    chunk_edges = _LANE * _CHUNK_ROWS
    e_pad = ((E + _NC * _NS * chunk_edges - 1)
             // (_NC * _NS * chunk_edges)) * (_NC * _NS * chunk_edges)
    nblk = e_pad // blk
    super_rows = e_pad // _LANE
    rows_per_worker = super_rows // (_NC * _NS)
    n_pad = ((N + _NS * 16 - 1) // (_NS * 16)) * (_NS * 16)

    w2 = jnp.concatenate([Ws2.reshape(1, D), Wi2.reshape(1, D)], axis=0)
    b2 = jnp.concatenate([bs2.reshape(1, 1), bi2.reshape(1, 1)], axis=1)

    nhalf = nblk // 2
    rows_half = super_rows // 2
    rows_per_worker = rows_half // (_NC * _NS)

    vt = edge_vec.T
    bs1r, bi1r = bs1.reshape(1, D), bi1.reshape(1, D)

    def stage1(off):
        return pl.pallas_call(
            functools.partial(_edge_body, nreal=nreal, off=off),
            grid=(nhalf,),
            in_specs=[
                pl.BlockSpec((blk, D), lambda i: (jnp.minimum(i + off, nreal - 1), 0)),
                pl.BlockSpec((3, blk), lambda i: (0, jnp.minimum(i + off, nreal - 1))),
                pl.BlockSpec((D, D), lambda i: (0, 0)),
                pl.BlockSpec((1, D), lambda i: (0, 0)),
                pl.BlockSpec((2, D), lambda i: (0, 0)),
                pl.BlockSpec((D, D), lambda i: (0, 0)),
                pl.BlockSpec((1, D), lambda i: (0, 0)),
                pl.BlockSpec((1, 2), lambda i: (0, 0)),
            ],
            out_specs=pl.BlockSpec((blk, 8), lambda i: (i, 0)),
            out_shape=jax.ShapeDtypeStruct((nhalf * blk, 8), jnp.float32),
        )(x_edge, vt, Ws1, bs1r, w2, Wi1, bi1r, b2)

    pad_n = e_pad - E
    idx_pad = jnp.concatenate(
        [idx_t, (jnp.arange(pad_n, dtype=jnp.int32) % N)])
    idx2 = idx_pad.reshape(super_rows, _LANE)
    zeros_acc = jnp.zeros((n_pad, 8), jnp.float32)

    mesh = plsc.VectorSubcoreMesh(core_axis_name="c", subcore_axis_name="s")
    scatter = pl.kernel(
        functools.partial(_scatter_body, n_pad=n_pad,
                          rows_per_worker=rows_per_worker),
        out_type=jax.ShapeDtypeStruct((_NC, n_pad, 8), jnp.float32),
        mesh=mesh,
        compiler_params=pltpu.CompilerParams(use_tc_tiling_on_sc=False),
        scratch_types=[
            pltpu.VMEM((_CHUNK_ROWS, _LANE), jnp.int32),
            pltpu.VMEM((_CHUNK_ROWS * _LANE, 8), jnp.float32),
            pltpu.VMEM_SHARED((n_pad, 8), jnp.float32),
            pltpu.SemaphoreType.DMA,
        ],
    )

    vals0 = stage1(0)
    p0 = scatter(vals0, idx2[:rows_half], zeros_acc)
    vals1 = stage1(nhalf)
    p1 = scatter(vals1, idx2[rows_half:], zeros_acc)

    bi_pad = jnp.concatenate(
        [batch_idx, jnp.full((n_pad - N,), B, jnp.int32)]).reshape(1, n_pad)
    cm = jnp.asarray(_change_mat_np())
    stress = pl.pallas_call(
        functools.partial(_finish_body, n_pad=n_pad, b=B),
        out_shape=jax.ShapeDtypeStruct((B, 9), jnp.float32),
    )(p0, p1, bi_pad, cm)
    return stress.reshape(B, 3, 3)


# speedup vs baseline: 1.4321x; 1.4255x over previous
import functools
import math

import jax
import jax.numpy as jnp
import numpy as np
from jax import lax
from jax.experimental import pallas as pl
from jax.experimental.pallas import tpu as pltpu
from jax.experimental.pallas import tpu_sc as plsc

_SQRT3 = math.sqrt(3.0)
_SH_NORM = math.sqrt(5.0 / (4.0 * math.pi))

_NC = 2
_NS = 16
_LANE = 128
_CHUNK_ROWS = 8
_CHUNK_E = _CHUNK_ROWS * _LANE


def _change_mat_np():
    s2 = 2 ** (-0.5)
    s3 = 3 ** (-0.5)
    s6 = 6 ** (-0.5)
    return np.array([
        [s3, 0, 0, 0, s3, 0, 0, 0, s3],
        [0, 0, 0, 0, 0, s2, 0, -s2, 0],
        [0, 0, -s2, 0, 0, 0, s2, 0, 0],
        [0, s2, 0, -s2, 0, 0, 0, 0, 0],
        [0, 0, 0.5 ** 0.5, 0, 0, 0, 0.5 ** 0.5, 0, 0],
        [0, s2, 0, s2, 0, 0, 0, 0, 0],
        [-s6, 0, 0, 0, 2 * s6, 0, 0, 0, -s6],
        [0, 0, 0, 0, 0, s2, 0, s2, 0],
        [-s2, 0, 0, 0, 0, 0, 0, 0, s2],
    ], dtype=np.float32)


def _edge_body(x_ref, vt_ref, ws1_ref, bs1_ref, w2_ref, wi1_ref, bi1_ref,
               b2_ref, o0, o1, o2, o3, o4, o5, *, e_total, off):
    x = x_ref[...]
    h1 = jnp.dot(x, ws1_ref[...], preferred_element_type=jnp.float32) + bs1_ref[...]
    h1 = h1 * (0.5 * jnp.tanh(0.5 * h1) + 0.5)
    es = jnp.sum(h1 * w2_ref[0:1, :], axis=1, keepdims=True) + b2_ref[0:1, 0:1]
    h2 = jnp.dot(x, wi1_ref[...], preferred_element_type=jnp.float32) + bi1_ref[...]
    h2 = h2 * (0.5 * jnp.tanh(0.5 * h2) + 0.5)
    ei = jnp.sum(h2 * w2_ref[1:2, :], axis=1, keepdims=True) + b2_ref[0:1, 1:2]
    esei_t = jnp.concatenate([es, ei], axis=1).T

    vt = vt_ref[...]
    blk_n = vt.shape[1]
    base = (pl.program_id(0) + off) * blk_n
    lanes = lax.broadcasted_iota(jnp.int32, (1, blk_n), 1)
    valid_b = (lanes + base) < e_total
    vx, vy, vz = vt[0:1, :], vt[1:2, :], vt[2:3, :]
    r = jnp.sqrt(vx * vx + vy * vy + vz * vz)
    rinv = 1.0 / jnp.maximum(r, 1e-12)
    ux, uy, uz = vx * rinv, vy * rinv, vz * rinv
    eis = esei_t[1:2, :] * _SH_NORM
    blk = vt.shape[1]
    zero = jnp.zeros_like(eis)

    def put(o_ref, v):
        o_ref[...] = jnp.where(valid_b, v, zero).reshape(blk)

    put(o0, esei_t[0:1, :])
    put(o1, (_SQRT3 * ux * uz) * eis)
    put(o2, (_SQRT3 * ux * uy) * eis)
    put(o3, (uy * uy - 0.5 * (ux * ux + uz * uz)) * eis)
    put(o4, (_SQRT3 * uy * uz) * eis)
    put(o5, (_SQRT3 / 2.0) * (uz * uz - ux * ux) * eis)


def _scatter_body(c0, c1, c2, c3, c4, c5, idx_hbm, zeros_hbm, valid_hbm,
                  out_hbm, idx_v, colv, vals_v, acc, sem, *,
                  n_pad, rows_per_worker):
    c = lax.axis_index("c")
    s = lax.axis_index("s")
    stripe = n_pad // _NS
    pltpu.sync_copy(zeros_hbm.at[pl.ds(s * stripe, stripe)],
                    acc.at[pl.ds(s * stripe, stripe)])
    iota16 = lax.iota(jnp.int32, 16)
    ones16 = jnp.ones((16,), jnp.float32)
    zeros16 = jnp.zeros((16,), jnp.float32)

    def fill(g, carry):
        rows = g * 16 + iota16
        plsc.store_scatter(vals_v, [rows, jnp.full((16,), 6, jnp.int32)], ones16)
        plsc.store_scatter(vals_v, [rows, jnp.full((16,), 7, jnp.int32)], zeros16)
        return carry

    lax.fori_loop(0, _CHUNK_E // 16, fill, 0)
    plsc.subcore_barrier()
    wid = c * _NS + s
    base = wid * rows_per_worker
    nchunks = rows_per_worker // _CHUNK_ROWS

    def chunk(i, carry):
        row = base + i * _CHUNK_ROWS
        e0 = row * _LANE
        pltpu.sync_copy(idx_hbm.at[pl.ds(row, _CHUNK_ROWS)], idx_v)
        for k, col in enumerate((c0, c1, c2, c3, c4, c5)):
            pltpu.sync_copy(col.at[pl.ds(e0, _CHUNK_E)], colv.at[k])
        pltpu.sync_copy(valid_hbm.at[pl.ds(e0, _CHUNK_E)], colv.at[6])

        def assemble(g, carry2):
            rows = g * 16 + iota16
            for k in range(7):
                x = colv[k, pl.ds(g * 16, 16)]
                plsc.store_scatter(
                    vals_v, [rows, jnp.full((16,), 6 if k == 6 else k,
                                            jnp.int32)], x)
            return carry2

        lax.fori_loop(0, _CHUNK_E // 16, assemble, 0)
        cps = [pltpu.async_copy(vals_v.at[pl.ds(j * _LANE, _LANE)],
                                acc.at[idx_v.at[j]], sem, add=True)
               for j in range(_CHUNK_ROWS)]
        for cp in cps:
            cp.wait()
        return carry

    lax.fori_loop(0, nchunks, chunk, 0)
    plsc.subcore_barrier()
    pltpu.sync_copy(acc.at[pl.ds(s * stripe, stripe)],
                    out_hbm.at[c, pl.ds(s * stripe, stripe)])


def _finish_body(p0_ref, p1_ref, bi_ref, cm_ref, out_ref, *, n_pad, b):
    accm = (p0_ref[0] + p0_ref[1]) + (p1_ref[0] + p1_ref[1])
    cnt = accm[:, 6:7]
    nv = accm[:, 0:6] / jnp.maximum(cnt, 1.0)
    ones = jnp.ones((n_pad, 1), jnp.float32)
    zeros = jnp.zeros((n_pad, 1), jnp.float32)
    nv8 = jnp.concatenate([nv, ones, zeros], axis=1)
    bi = bi_ref[...]
    rows = lax.broadcasted_iota(jnp.int32, (b, n_pad), 0)
    oh = (rows == bi).astype(jnp.float32)
    seg = jnp.dot(oh, nv8, preferred_element_type=jnp.float32)
    nb = jnp.maximum(seg[:, 6:7], 1.0)
    g = seg[:, 0:6] / nb
    flat = jnp.concatenate(
        [g[:, 0:1], jnp.zeros((b, 3), jnp.float32), g[:, 1:6]], axis=1)
    out_ref[...] = jnp.dot(flat, cm_ref[...], preferred_element_type=jnp.float32)


def kernel(x_edge, edge_vec, idx_t, batch_idx, batch_size,
           Ws1, bs1, Ws2, bs2, Wi1, bi1, Wi2, bi2):
    E, D = x_edge.shape
    N = batch_idx.shape[0]
    B = 16

    blk = 2048
    nreal = (E + blk - 1) // blk - 1
    e_pad = ((E + _NC * _NS * _CHUNK_E - 1)
             // (_NC * _NS * _CHUNK_E)) * (_NC * _NS * _CHUNK_E)
    nblk = e_pad // blk
    super_rows = e_pad // _LANE
    n_pad = ((N + _NS * 16 - 1) // (_NS * 16)) * (_NS * 16)

    w2 = jnp.concatenate([Ws2.reshape(1, D), Wi2.reshape(1, D)], axis=0)
    b2 = jnp.concatenate([bs2.reshape(1, 1), bi2.reshape(1, 1)], axis=1)

    nhalf = nblk // 2
    e_half = nhalf * blk
    rows_half = super_rows // 2
    rows_per_worker = rows_half // (_NC * _NS)

    vt = edge_vec.T
    bs1r, bi1r = bs1.reshape(1, D), bi1.reshape(1, D)
    col_ty = jax.ShapeDtypeStruct((e_half,), jnp.float32)

    def stage1(off):
        return pl.pallas_call(
            functools.partial(_edge_body, e_total=E, off=off),
            grid=(nhalf,),
            in_specs=[
                pl.BlockSpec((blk, D), lambda i: (jnp.minimum(i + off, nreal), 0)),
                pl.BlockSpec((3, blk), lambda i: (0, jnp.minimum(i + off, nreal))),
                pl.BlockSpec((D, D), lambda i: (0, 0)),
                pl.BlockSpec((1, D), lambda i: (0, 0)),
                pl.BlockSpec((2, D), lambda i: (0, 0)),
                pl.BlockSpec((D, D), lambda i: (0, 0)),
                pl.BlockSpec((1, D), lambda i: (0, 0)),
                pl.BlockSpec((1, 2), lambda i: (0, 0)),
            ],
            out_specs=[pl.BlockSpec((blk,), lambda i: (i,))] * 6,
            out_shape=[col_ty] * 6,
        )(x_edge, vt, Ws1, bs1r, w2, Wi1, bi1r, b2)

    pad_n = e_pad - E
    idx_pad = jnp.concatenate(
        [idx_t, (jnp.arange(pad_n, dtype=jnp.int32) % N)])
    idx2 = idx_pad.reshape(super_rows, _LANE)
    valid_e = jnp.concatenate(
        [jnp.ones((E,), jnp.float32), jnp.zeros((pad_n,), jnp.float32)])
    zeros_acc = jnp.zeros((n_pad, 8), jnp.float32)

    mesh = plsc.VectorSubcoreMesh(core_axis_name="c", subcore_axis_name="s")
    scatter = pl.kernel(
        functools.partial(_scatter_body, n_pad=n_pad,
                          rows_per_worker=rows_per_worker),
        out_type=jax.ShapeDtypeStruct((_NC, n_pad, 8), jnp.float32),
        mesh=mesh,
        compiler_params=pltpu.CompilerParams(use_tc_tiling_on_sc=False,
                                             needs_layout_passes=False),
        scratch_types=[
            pltpu.VMEM((_CHUNK_ROWS, _LANE), jnp.int32),
            pltpu.VMEM((7, _CHUNK_E), jnp.float32),
            pltpu.VMEM((_CHUNK_E, 8), jnp.float32),
            pltpu.VMEM_SHARED((n_pad, 8), jnp.float32),
            pltpu.SemaphoreType.DMA,
        ],
    )

    cols0 = stage1(0)
    p0 = scatter(*cols0, idx2[:rows_half], zeros_acc, valid_e[:e_half])
    cols1 = stage1(nhalf)
    p1 = scatter(*cols1, idx2[rows_half:], zeros_acc, valid_e[e_half:])

    bi_pad = jnp.concatenate(
        [batch_idx, jnp.full((n_pad - N,), B, jnp.int32)]).reshape(1, n_pad)
    cm = jnp.asarray(_change_mat_np())
    stress = pl.pallas_call(
        functools.partial(_finish_body, n_pad=n_pad, b=B),
        out_shape=jax.ShapeDtypeStruct((B, 9), jnp.float32),
    )(p0, p1, bi_pad, cm)
    return stress.reshape(B, 3, 3)
